# baseline (speedup 1.0000x reference)
import functools

import jax
import jax.numpy as jnp
from jax import lax
from jax.experimental import pallas as pl
from jax.experimental.pallas import tpu as pltpu
from jax.experimental.pallas import tpu_sc as plsc

NUM_PATCHES = 576
PROJECTION_DIM = 768
BATCH = 64

NC = 2
NS = 16
NW = NC * NS
PW = NUM_PATCHES // NW
CHUNK = PW * PROJECTION_DIM
ROW = NUM_PATCHES * PROJECTION_DIM
TOTAL = BATCH * ROW
LANES = 16
NVEC = CHUNK // LANES


def _sc_add(x_hbm, pos_hbm, out_hbm, pos_v, x_v, o_v):
    wid = lax.axis_index("s") * NC + lax.axis_index("c")
    base = wid * CHUNK
    pltpu.sync_copy(pos_hbm.at[pl.ds(base, CHUNK)], pos_v)

    def batch_body(b, _):
        off = b * ROW + base
        pltpu.sync_copy(x_hbm.at[pl.ds(off, CHUNK)], x_v)

        def vec_body(i, _):
            sl = pl.ds(i * LANES, LANES)
            o_v[sl] = x_v[sl] + pos_v[sl]
            return ()

        lax.fori_loop(0, NVEC, vec_body, ())
        pltpu.sync_copy(o_v, out_hbm.at[pl.ds(off, CHUNK)])
        return ()

    lax.fori_loop(0, BATCH, batch_body, ())


def kernel(x, pos_table):
    mesh = plsc.VectorSubcoreMesh(core_axis_name="c", subcore_axis_name="s")
    run = functools.partial(
        pl.kernel,
        out_type=jax.ShapeDtypeStruct((TOTAL,), jnp.float32),
        mesh=mesh,
        scratch_types=[
            pltpu.VMEM((CHUNK,), jnp.float32),
            pltpu.VMEM((CHUNK,), jnp.float32),
            pltpu.VMEM((CHUNK,), jnp.float32),
        ],
    )(_sc_add)
    out = run(x.reshape(-1), pos_table.reshape(-1))
    return out.reshape(x.shape)
